# Initial kernel scaffold; baseline (speedup 1.0000x reference)
import jax
import jax.numpy as jnp
from jax.experimental import pallas as pl


def kernel(x, r0_conv0_w, r0_conv0_b, r0_conv1_w, r0_conv1_b, r0_conv2_w, r0_conv2_b, r0_conv3_w, r0_conv3_b, r0_conv4_w, r0_conv4_b, r0_post_w, r0_post_b, r0_basis, r1_conv0_w, r1_conv0_b, r1_conv1_w, r1_conv1_b, r1_conv2_w, r1_conv2_b, r1_conv3_w, r1_conv3_b, r1_conv4_w, r1_conv4_b, r1_post_w, r1_post_b, r1_basis, r2_conv0_w, r2_conv0_b, r2_conv1_w, r2_conv1_b, r2_conv2_w, r2_conv2_b, r2_conv3_w, r2_conv3_b, r2_conv4_w, r2_conv4_b, r2_post_w, r2_post_b, r2_basis):
    raise NotImplementedError("write your pallas kernel here")


# speedup vs baseline: 6.5058x; 6.5058x over previous
import functools

import jax
import jax.numpy as jnp
from jax.experimental import pallas as pl
from jax.experimental.pallas import tpu as pltpu


def _rup(x, m):
    return ((x + m - 1) // m) * m


def _dft_mag_body(fr_ref, basis_ref, o_ref):
    acc = jnp.dot(fr_ref[...], basis_ref[...], preferred_element_type=jnp.float32)
    re = acc[:, :128]
    im = acc[:, 128:]
    o_ref[...] = jnp.sqrt(re * re + im * im).astype(o_ref.dtype)


def _stft_mag(x, basis, n_fft, hop):
    B, T = x.shape
    nb = basis.shape[1] // 256
    F_pad = nb * 128
    n_freq = n_fft // 2 + 1

    pad = n_fft // 2
    xp = jnp.pad(x, ((0, 0), (pad, pad)), mode="reflect")
    n_frames = 1 + T // hop
    ratio = n_fft // hop
    n_chunks = n_frames - 1 + ratio
    xs = xp[:, :n_chunks * hop].reshape(B, n_chunks, hop)
    frames = jnp.concatenate([xs[:, j:j + n_frames, :] for j in range(ratio)],
                             axis=-1)
    frames = frames.reshape(B * n_frames, n_fft).astype(jnp.bfloat16)

    M = B * n_frames
    M_blk = min(1024, _rup(M, 256))
    M_pad = _rup(M, M_blk)
    frames = jnp.pad(frames, ((0, M_pad - M), (0, 0)))

    out = pl.pallas_call(
        _dft_mag_body,
        out_shape=jax.ShapeDtypeStruct((M_pad, F_pad), jnp.bfloat16),
        grid=(M_pad // M_blk, nb),
        in_specs=[
            pl.BlockSpec((M_blk, n_fft), lambda mb, fb: (mb, 0)),
            pl.BlockSpec((n_fft, 256), lambda mb, fb: (0, fb)),
        ],
        out_specs=pl.BlockSpec((M_blk, 128), lambda mb, fb: (mb, fb)),
        compiler_params=pltpu.CompilerParams(
            dimension_semantics=("parallel", "parallel")),
    )(frames, basis)
    return out[:M, :n_freq].reshape(B, n_frames, n_freq)


def _conv_body(*refs, KH, KW, sh, Wc, n_wi, Hp, Cs, n_ci, Cb):
    x_refs = refs[:n_ci]
    w_ref, b_ref, o_ref, acc_ref = refs[n_ci:]
    M = Wc * Hp
    Ks = KW * Cs
    stop = sh * (Hp - 1) + 1
    for wi in range(n_wi):
        w0 = wi * Wc
        first = True
        for s in range(n_ci):
            for kh in range(KH):
                parts = [
                    x_refs[s][0, w0 + kw:w0 + kw + Wc, kh:kh + stop:sh, :]
                    for kw in range(KW)
                ]
                tap = (jnp.concatenate(parts, axis=-1) if KW > 1 else parts[0])
                tap = tap.astype(jnp.bfloat16).reshape(M, Ks)
                contrib = jnp.dot(tap, w_ref[kh, s * Ks:(s + 1) * Ks, :],
                                  preferred_element_type=jnp.float32)
                if first:
                    acc_ref[...] = contrib
                    first = False
                else:
                    acc_ref[...] += contrib
        o_ref[0, w0:w0 + Wc] = (
            (acc_ref[...] + b_ref[...]).astype(o_ref.dtype).reshape(Wc, Hp, Cb))


def _conv_call(xs, wt, bt, KH, KW, sh, W_out, H_out_p, H_blk, Cs, C_out_pad):
    n_ci = len(xs)
    B, W_pad = xs[0].shape[0], xs[0].shape[1]
    max_wc = max(1, 1024 // H_out_p)
    n_wi = pl.cdiv(W_out, max_wc)
    Wc = pl.cdiv(W_out, n_wi)
    Wp = Wc * n_wi
    assert W_pad >= Wp + KW - 1
    body = functools.partial(_conv_body, KH=KH, KW=KW, sh=sh, Wc=Wc,
                             n_wi=n_wi, Hp=H_out_p, Cs=Cs, n_ci=n_ci, Cb=128)
    return pl.pallas_call(
        body,
        out_shape=jax.ShapeDtypeStruct((B, Wp, H_out_p, C_out_pad), jnp.bfloat16),
        grid=(B, C_out_pad // 128),
        in_specs=[pl.BlockSpec((1, W_pad, H_blk, Cs),
                               lambda bb, ci: (bb, 0, 0, 0))] * n_ci + [
            pl.BlockSpec((KH, n_ci * KW * Cs, 128), lambda bb, ci: (0, 0, ci)),
            pl.BlockSpec((1, 128), lambda bb, ci: (0, ci)),
        ],
        out_specs=pl.BlockSpec((1, Wp, H_out_p, 128),
                               lambda bb, ci: (bb, 0, 0, ci)),
        scratch_shapes=[pltpu.VMEM((Wc * H_out_p, 128), jnp.float32)],
        compiler_params=pltpu.CompilerParams(
            dimension_semantics=("parallel", "parallel"),
            vmem_limit_bytes=100 * 1024 * 1024),
    )(*xs, wt, bt)


def _conv_layer(x, valid_W, valid_H, w, b, sh, ph, pw):
    C_out, C_in, KH, KW = w.shape
    H_out = (valid_H + 2 * ph - KH) // sh + 1
    W_out = valid_W + 2 * pw - KW + 1
    H_out_p = _rup(H_out, 16)
    H_blk = _rup(max(KH + sh * (H_out_p - 1), valid_H + ph), 16)
    C_out_pad = _rup(C_out, 128)

    max_wc = max(1, 1024 // H_out_p)
    n_wi = pl.cdiv(W_out, max_wc)
    Wc = pl.cdiv(W_out, n_wi)
    W_pad = Wc * n_wi + KW - 1

    n_ci = pl.cdiv(C_in, 128)
    Cs = C_in // n_ci
    xs = []
    for s in range(n_ci):
        xv = x[:, :valid_W, :valid_H, s * Cs:(s + 1) * Cs].astype(jnp.float32)
        xs.append(jnp.pad(xv, ((0, 0), (pw, W_pad - pw - valid_W),
                               (ph, H_blk - ph - valid_H), (0, 0))))

    wt4 = jnp.transpose(w, (2, 3, 1, 0))
    wt = jnp.concatenate(
        [wt4[:, :, s * Cs:(s + 1) * Cs, :].reshape(KH, KW * Cs, C_out)
         for s in range(n_ci)], axis=1)
    wt = jnp.pad(wt, ((0, 0), (0, 0), (0, C_out_pad - C_out))).astype(jnp.bfloat16)
    bt = jnp.pad(b, (0, C_out_pad - C_out)).reshape(1, C_out_pad).astype(jnp.float32)

    out = _conv_call(xs, wt, bt, KH, KW, sh, W_out, H_out_p, H_blk, Cs, C_out_pad)
    return out, W_out, H_out


def _conv0_layer(spec, w, b, sh, ph, pw):
    B, W_in, H_in = spec.shape
    C_out, _, KH, KW = w.shape
    H_out = (H_in + 2 * ph - KH) // sh + 1
    W_out = W_in + 2 * pw - KW + 1
    H_out_p = _rup(H_out, 16)
    C_out_pad = _rup(C_out, 128)
    KC = KH * KW
    KC_pad = _rup(KC, 8)

    H_need = (H_out_p - 1) * sh + KH
    xp = jnp.pad(spec, ((0, 0), (pw, pw),
                        (ph, max(0, H_need - H_in - ph))))
    cols = []
    for kh in range(KH):
        for kw in range(KW):
            cols.append(xp[:, kw:kw + W_out, kh:kh + sh * H_out_p:sh, None])
    xk = jnp.concatenate(cols, axis=-1)
    if KC_pad != KC:
        xk = jnp.pad(xk, ((0, 0), (0, 0), (0, 0), (0, KC_pad - KC)))
    xk = xk.astype(jnp.float32)

    wt = jnp.transpose(w, (2, 3, 1, 0)).reshape(KC, C_out)
    wt = jnp.pad(wt, ((0, KC_pad - KC), (0, C_out_pad - C_out)))
    wt = wt.reshape(1, KC_pad, C_out_pad).astype(jnp.bfloat16)
    bt = jnp.pad(b, (0, C_out_pad - C_out)).reshape(1, C_out_pad).astype(jnp.float32)

    max_wc = max(1, 1024 // H_out_p)
    n_wi = pl.cdiv(W_out, max_wc)
    Wc = pl.cdiv(W_out, n_wi)
    W_pad = Wc * n_wi
    if W_pad != W_out:
        xk = jnp.pad(xk, ((0, 0), (0, W_pad - W_out), (0, 0), (0, 0)))

    out = _conv_call([xk], wt, bt, 1, 1, 1, W_out, H_out_p, H_out_p, KC_pad,
                     C_out_pad)
    return out, W_out, H_out


def _disc_r(x, convs, post, basis, resolution):
    hop = resolution
    n_fft = resolution * 4
    spec = _stft_mag(x, basis, n_fft, hop)

    feats = []
    (w0, b0) = convs[0]
    h, vW, vH = _conv0_layer(spec, w0, b0, 2, 3, 1)
    feats.append((h, vW, vH, w0.shape[0]))
    for (w, b) in convs[1:]:
        h, vW, vH = _conv_layer(h, vW, vH, w, b, 2, 2, 1)
        feats.append((h, vW, vH, w.shape[0]))
    wp, bp = post
    h, vW, vH = _conv_layer(h, vW, vH, wp, bp, 1, 1, 1)
    feats.append((h, vW, vH, wp.shape[0]))

    feats_nchw = [
        jnp.transpose(f[:, :vw, :vh, :c], (0, 3, 2, 1)).astype(jnp.float32)
        for (f, vw, vh, c) in feats
    ]
    return feats_nchw[-1], feats_nchw


def kernel(x, r0_conv0_w, r0_conv0_b, r0_conv1_w, r0_conv1_b, r0_conv2_w, r0_conv2_b, r0_conv3_w, r0_conv3_b, r0_conv4_w, r0_conv4_b, r0_post_w, r0_post_b, r0_basis, r1_conv0_w, r1_conv0_b, r1_conv1_w, r1_conv1_b, r1_conv2_w, r1_conv2_b, r1_conv3_w, r1_conv3_b, r1_conv4_w, r1_conv4_b, r1_post_w, r1_post_b, r1_basis, r2_conv0_w, r2_conv0_b, r2_conv1_w, r2_conv1_b, r2_conv2_w, r2_conv2_b, r2_conv3_w, r2_conv3_b, r2_conv4_w, r2_conv4_b, r2_post_w, r2_post_b, r2_basis):
    groups = [
        (128, [(r0_conv0_w, r0_conv0_b), (r0_conv1_w, r0_conv1_b),
               (r0_conv2_w, r0_conv2_b), (r0_conv3_w, r0_conv3_b),
               (r0_conv4_w, r0_conv4_b)], (r0_post_w, r0_post_b), r0_basis),
        (256, [(r1_conv0_w, r1_conv0_b), (r1_conv1_w, r1_conv1_b),
               (r1_conv2_w, r1_conv2_b), (r1_conv3_w, r1_conv3_b),
               (r1_conv4_w, r1_conv4_b)], (r1_post_w, r1_post_b), r1_basis),
        (512, [(r2_conv0_w, r2_conv0_b), (r2_conv1_w, r2_conv1_b),
               (r2_conv2_w, r2_conv2_b), (r2_conv3_w, r2_conv3_b),
               (r2_conv4_w, r2_conv4_b)], (r2_post_w, r2_post_b), r2_basis),
    ]
    logits, feats = [], []
    for (res, convs, post, basis) in groups:
        logit, fmap = _disc_r(x, convs, post, basis, res)
        logits.append(logit)
        feats += fmap
    return logits, feats
